# baseline (speedup 1.0000x reference)
import jax
import jax.numpy as jnp
from jax import lax
from jax.experimental import pallas as pl
from jax.experimental.pallas import tpu as pltpu
from jax.experimental.pallas import tpu_sc as plsc

_N = 10000
_E = 320000
_D = 128
_NC = 2
_NS = 16
_NW = _NC * _NS
_EPW = _E // _NW
_CH = 80
_CHB = 128
_NCHUNK = _EPW // _CH
_NP = 10240
_NPASS = 5
_NPH = 2048
_NPA = _NPASS * _NPH
_RPS = _NPH // _NS
_DR = _NP // _D
_BLK = 2000


def _tcmain_body(p_ref, x_ref, n0_ref, n1_ref, d0_ref, d1_ref,
                 w_ref, b_ref, av_ref, adv_ref,
                 xp_ref, als_ref, ald_ref):
    num = n0_ref[...] + n1_ref[...]
    den = d0_ref[...] + d1_ref[...]
    hprev = jnp.maximum(num / (den + 1e-16), 0.0)
    h = jnp.where(p_ref[0, 0] > 0.0, x_ref[...], hprev)
    xp = jnp.dot(h, w_ref[...],
                 preferred_element_type=jnp.float32) + b_ref[...]
    xp_ref[...] = xp
    als_ref[...] = jnp.dot(xp, av_ref[...], preferred_element_type=jnp.float32)
    ald_ref[...] = jnp.dot(xp, adv_ref[...], preferred_element_type=jnp.float32)


def _tc3_body(n0_ref, n1_ref, d0_ref, d1_ref, o_ref):
    num = n0_ref[...] + n1_ref[...]
    den = d0_ref[...] + d1_ref[...]
    o_ref[...] = jnp.maximum(num / (den + 1e-16), 0.0)


_row_spec = pl.BlockSpec((_BLK, _D), lambda i: (i, 0))
_one_spec = pl.BlockSpec((_BLK, 1), lambda i: (i, 0))

_tcmain = pl.pallas_call(
    _tcmain_body,
    grid=(_N // _BLK,),
    in_specs=[
        pl.BlockSpec((1, 1), lambda i: (0, 0)),
        _row_spec, _row_spec, _row_spec, _one_spec, _one_spec,
        pl.BlockSpec((_D, _D), lambda i: (0, 0)),
        pl.BlockSpec((1, _D), lambda i: (0, 0)),
        pl.BlockSpec((_D, 1), lambda i: (0, 0)),
        pl.BlockSpec((_D, 1), lambda i: (0, 0)),
    ],
    out_specs=[_row_spec, _one_spec, _one_spec],
    out_shape=[
        jax.ShapeDtypeStruct((_NP, _D), jnp.float32),
        jax.ShapeDtypeStruct((_NP, 1), jnp.float32),
        jax.ShapeDtypeStruct((_NP, 1), jnp.float32),
    ],
)

_dense3 = pl.pallas_call(
    _tc3_body,
    grid=(_N // _BLK,),
    in_specs=[_row_spec, _row_spec, _one_spec, _one_spec],
    out_specs=_row_spec,
    out_shape=jax.ShapeDtypeStruct((_N, _D), jnp.float32),
)


def _seg_totals(k, v):
    i32 = jnp.int32
    lane = lax.iota(i32, 16)
    prev = k.at[jnp.maximum(lane - 1, 0)].get(mode="promise_in_bounds")
    nxt = k.at[jnp.minimum(lane + 1, 15)].get(mode="promise_in_bounds")
    m_start = (k != prev) | (lane == 0)
    m_last = (k != nxt) | (lane == 15)
    c = plsc.cumsum(v)
    ec = c - v
    ff = plsc.cummax(jnp.where(m_start, ec, 0.0))
    return c - ff, m_last


def _edge_body(xp_hbm, ei_hbm, als_hbm, ald_hbm,
               num_hbm, den_hbm,
               src_v, dst_v, als_v, ald_v, rows2_v, srows_v, exb_v,
               srcb_v, dstl_v, den_v, idx_v, bkt_v, acc_sh, den_sh,
               sem0, sem1):
    cc_ = lax.axis_index("c")
    s = lax.axis_index("s")
    w = s * _NC + cc_

    pltpu.sync_copy(ei_hbm.at[pl.ds(w * _EPW, _EPW)], src_v)
    pltpu.sync_copy(ei_hbm.at[pl.ds(_E + w * _EPW, _EPW)], dst_v)
    pltpu.sync_copy(als_hbm, als_v)
    pltpu.sync_copy(ald_hbm, ald_v)

    zeros16 = jnp.zeros((16,), jnp.float32)
    izeros16 = jnp.zeros((16,), jnp.int32)

    def zrow(r, carry):
        for k in range(_D // 16):
            srows_v[r, pl.ds(k * 16, 16)] = zeros16
        return carry

    lax.fori_loop(0, 64, zrow, 0)

    def zden(r, carry):
        for k in range(_D // 16):
            den_v[r, pl.ds(k * 16, 16)] = zeros16
        return carry

    lax.fori_loop(0, _DR, zden, 0)

    for g in range(_DR // 16):
        idx_v[pl.ds(g * 16, 16)] = lax.iota(jnp.int32, 16) + g * 16

    for g in range(3 * _CHB // 16):
        bkt_v[pl.ds(_EPW + g * 16, 16)] = izeros16

    for t in range(_RPS // 64):
        pltpu.sync_copy(srows_v, acc_sh.at[pl.ds(s * _RPS + t * 64, 64)])

    @pl.when(s < _DR // 8)
    def _zero_den_sh():
        pltpu.sync_copy(srows_v.at[pl.ds(0, 8)], den_sh.at[pl.ds(s * 8, 8)])

    plsc.subcore_barrier()

    def cntf(j, c):
        eb = j * _CH
        c0, c1, c2, c3, c4 = c
        for g in range(_CH // 16):
            di = dst_v[pl.ds(eb + g * 16, 16)]
            b16 = lax.shift_right_logical(di, 11)
            c0 += plsc.all_reduce_population_count(b16 == 0)[0]
            c1 += plsc.all_reduce_population_count(b16 == 1)[0]
            c2 += plsc.all_reduce_population_count(b16 == 2)[0]
            c3 += plsc.all_reduce_population_count(b16 == 3)[0]
            c4 += plsc.all_reduce_population_count(b16 == 4)[0]
        return (c0, c1, c2, c3, c4)

    zi = jnp.int32(0)
    c0, c1, c2, c3, c4 = lax.fori_loop(0, _NCHUNK, cntf,
                                       (zi, zi, zi, zi, zi))
    o1 = c0
    o2 = o1 + c1
    o3 = o2 + c2
    o4 = o3 + c3

    def appf(j, pos):
        eb = j * _CH
        p0, p1, p2, p3, p4 = pos
        for g in range(_CH // 16):
            si = src_v[pl.ds(eb + g * 16, 16)]
            di = dst_v[pl.ds(eb + g * 16, 16)]
            a_s = plsc.load_gather(als_v, [si])
            a_d = plsc.load_gather(ald_v, [di])
            e = a_s + a_d
            e = jnp.where(e >= 0.0, e, 0.2 * e)
            ex = jnp.exp(e)
            dk, dv = plsc.sort_key_val(di, ex)
            tot, m_last = _seg_totals(dk, dv)
            plsc.addupdate_scatter(
                den_v,
                [lax.shift_right_logical(dk, 7),
                 lax.bitwise_and(dk, jnp.full((16,), 127, jnp.int32))],
                tot, mask=m_last)
            b16 = lax.shift_right_logical(di, 11)
            pk = si * 4096 + (di - b16 * _NPH)
            m0 = b16 == 0
            m1 = b16 == 1
            m2 = b16 == 2
            m3 = b16 == 3
            m4 = b16 == 4
            plsc.store_compressed(bkt_v.at[pl.ds(p0, 16)], pk, mask=m0)
            plsc.store_compressed(bkt_v.at[pl.ds(p1, 16)], pk, mask=m1)
            plsc.store_compressed(bkt_v.at[pl.ds(p2, 16)], pk, mask=m2)
            plsc.store_compressed(bkt_v.at[pl.ds(p3, 16)], pk, mask=m3)
            plsc.store_compressed(bkt_v.at[pl.ds(p4, 16)], pk, mask=m4)
            p0 += plsc.all_reduce_population_count(m0)[0]
            p1 += plsc.all_reduce_population_count(m1)[0]
            p2 += plsc.all_reduce_population_count(m2)[0]
            p3 += plsc.all_reduce_population_count(m3)[0]
            p4 += plsc.all_reduce_population_count(m4)[0]
        return (p0, p1, p2, p3, p4)

    lax.fori_loop(0, _NCHUNK, appf, (zi, o1, o2, o3, o4))

    pltpu.sync_copy(den_v, den_sh.at[idx_v], add=True)
    plsc.subcore_barrier()

    @pl.when(s < _DR // 8)
    def _write_den():
        dsl = pl.ds(s * 8, 8)
        pltpu.sync_copy(den_sh.at[dsl], den_hbm.at[cc_, dsl])

    lax.fori_loop(0, _DR, zden, 0)

    starts = (zi, o1, o2, o3, o4)
    counts = (c0, c1, c2, c3, c4)
    mask4095 = jnp.full((16,), 4095, jnp.int32)

    for p in range(_NPASS):
        base = p * _NPH
        off = starts[p]
        cnt = counts[p]
        npairs = (cnt + 2 * _CHB - 1) // (2 * _CHB)

        def stage(jj, b):
            cb = off + jj * _CHB

            def sg(g, carry):
                pk = bkt_v[pl.ds(cb + g * 16, 16)]
                si = lax.shift_right_logical(pk, 12)
                dl = lax.bitwise_and(pk, mask4095)
                a_s = plsc.load_gather(als_v, [si])
                a_d = plsc.load_gather(ald_v, [dl + base])
                e = a_s + a_d
                e = jnp.where(e >= 0.0, e, 0.2 * e)
                ex = jnp.exp(e)
                lane = lax.iota(jnp.int32, 16) + (jj * _CHB + g * 16)
                exb_v[b, pl.ds(g * 16, 16)] = jnp.where(lane < cnt, ex, 0.0)
                srcb_v[b, pl.ds(g * 16, 16)] = si
                dstl_v[b, pl.ds(g * 16, 16)] = dl
                return carry

            lax.fori_loop(0, _CHB // 16, sg, 0)

        def gwait(b, sem_b):
            pltpu.make_async_copy(xp_hbm.at[srcb_v.at[b]],
                                  rows2_v.at[b], sem_b).wait()

        def proc(b):
            def sgroup(g2, icarry):
                exv = exb_v[b, pl.ds(g2 * 16, 16)]
                for l in range(16):
                    sc = exv[l]
                    r = g2 * 16 + l
                    for k in range(_D // 16):
                        rows2_v[b, r, pl.ds(k * 16, 16)] = (
                            rows2_v[b, r, pl.ds(k * 16, 16)] * sc)
                return icarry

            lax.fori_loop(0, _CHB // 16, sgroup, 0)
            pltpu.sync_copy(rows2_v.at[b], acc_sh.at[dstl_v.at[b]], add=True)

        stage(0, 0)
        pltpu.async_copy(xp_hbm.at[srcb_v.at[0]], rows2_v.at[0], sem0)

        def pair(j2, carry):
            je = 2 * j2
            stage(je + 1, 1)
            pltpu.async_copy(xp_hbm.at[srcb_v.at[1]], rows2_v.at[1], sem1)
            gwait(0, sem0)
            proc(0)
            stage(je + 2, 0)
            pltpu.async_copy(xp_hbm.at[srcb_v.at[0]], rows2_v.at[0], sem0)
            gwait(1, sem1)
            proc(1)
            return carry

        lax.fori_loop(0, npairs, pair, 0)
        gwait(0, sem0)
        plsc.subcore_barrier()

        for t in range(_RPS // 64):
            sl = pl.ds(s * _RPS + t * 64, 64)
            osl = pl.ds(base + s * _RPS + t * 64, 64)
            pltpu.sync_copy(acc_sh.at[sl], num_hbm.at[cc_, osl])
        if p < _NPASS - 1:
            for t in range(_RPS // 64):
                sl = pl.ds(s * _RPS + t * 64, 64)
                pltpu.sync_copy(den_v.at[pl.ds(0, 64)], acc_sh.at[sl])
            plsc.subcore_barrier()


_edge = pl.kernel(
    _edge_body,
    out_type=[
        jax.ShapeDtypeStruct((_NC, _NPA, _D), jnp.float32),
        jax.ShapeDtypeStruct((_NC, _DR, _D), jnp.float32),
    ],
    mesh=plsc.VectorSubcoreMesh(core_axis_name="c", subcore_axis_name="s"),
    compiler_params=pltpu.CompilerParams(needs_layout_passes=False),
    scratch_types=[
        pltpu.VMEM((_EPW,), jnp.int32),
        pltpu.VMEM((_EPW,), jnp.int32),
        pltpu.VMEM((_NP,), jnp.float32),
        pltpu.VMEM((_NP,), jnp.float32),
        pltpu.VMEM((2, _CHB, _D), jnp.float32),
        pltpu.VMEM((64, _D), jnp.float32),
        pltpu.VMEM((2, _CHB), jnp.float32),
        pltpu.VMEM((2, _CHB), jnp.int32),
        pltpu.VMEM((2, _CHB), jnp.int32),
        pltpu.VMEM((_DR, _D), jnp.float32),
        pltpu.VMEM((_DR,), jnp.int32),
        pltpu.VMEM((_EPW + 3 * _CHB,), jnp.int32),
        pltpu.VMEM_SHARED((_NPH, _D), jnp.float32),
        pltpu.VMEM_SHARED((_DR, _D), jnp.float32),
        pltpu.SemaphoreType.DMA,
        pltpu.SemaphoreType.DMA,
    ],
)


def kernel(x, edge_index, W1, b1, as1, ad1, Wk1, bk1, q1,
           W2, b2, as2, ad2, Wk2, bk2, q2):
    ei_flat = edge_index.reshape(-1)
    ws = jnp.stack([W1, W2])
    bs = jnp.stack([b1.reshape(1, _D), b2.reshape(1, _D)])
    avs = jnp.stack([as1.reshape(_D, 1), as2.reshape(_D, 1)])
    advs = jnp.stack([ad1.reshape(_D, 1), ad2.reshape(_D, 1)])
    preds = jnp.array([[[1.0]], [[0.0]]], dtype=jnp.float32)

    num0 = jnp.zeros((_NC, _NPA, _D), jnp.float32)
    den0 = jnp.zeros((_NC, _DR, _D), jnp.float32)

    def body(carry, layer):
        num, den = carry
        pred, w, b, av, adv = layer
        xp, als, ald = _tcmain(pred, x, num[0], num[1],
                               den[0].reshape(_NP, 1),
                               den[1].reshape(_NP, 1),
                               w, b, av, adv)
        num2, den2 = _edge(xp, ei_flat, als.reshape(-1), ald.reshape(-1))
        return (num2, den2), 0

    (num_f, den_f), _ = lax.scan(body, (num0, den0),
                                 (preds, ws, bs, avs, advs))
    return _dense3(num_f[0], num_f[1],
                   den_f[0].reshape(_NP, 1), den_f[1].reshape(_NP, 1))


# speedup vs baseline: 1.2315x; 1.2315x over previous
import jax
import jax.numpy as jnp
from jax import lax
from jax.experimental import pallas as pl
from jax.experimental.pallas import tpu as pltpu
from jax.experimental.pallas import tpu_sc as plsc

_N = 10000
_E = 320000
_D = 128
_NC = 2
_NS = 16
_NW = _NC * _NS
_EPW = _E // _NW
_CH = 80
_CHB = 80
_NCHUNK = _EPW // _CH
_NP = 10240
_NPASS = 5
_NPH = 2048
_NPA = _NPASS * _NPH
_RPS = _NPH // _NS
_DR = _NP // _D
_BLK = 2000


def _tcmain_body(p_ref, x_ref, n0_ref, n1_ref, d0_ref, d1_ref,
                 w_ref, b_ref, av_ref, adv_ref,
                 xp_ref, als_ref, ald_ref):
    num = n0_ref[...] + n1_ref[...]
    den = d0_ref[...] + d1_ref[...]
    hprev = jnp.maximum(num / (den + 1e-16), 0.0)
    h = jnp.where(p_ref[0, 0] > 0.0, x_ref[...], hprev)
    xp = jnp.dot(h, w_ref[...],
                 preferred_element_type=jnp.float32) + b_ref[...]
    xp_ref[...] = xp
    als_ref[...] = jnp.dot(xp, av_ref[...], preferred_element_type=jnp.float32)
    ald_ref[...] = jnp.dot(xp, adv_ref[...], preferred_element_type=jnp.float32)


def _tc3_body(n0_ref, n1_ref, d0_ref, d1_ref, o_ref):
    num = n0_ref[...] + n1_ref[...]
    den = d0_ref[...] + d1_ref[...]
    o_ref[...] = jnp.maximum(num / (den + 1e-16), 0.0)


_row_spec = pl.BlockSpec((_BLK, _D), lambda i: (i, 0))
_one_spec = pl.BlockSpec((_BLK, 1), lambda i: (i, 0))

_tcmain = pl.pallas_call(
    _tcmain_body,
    grid=(_N // _BLK,),
    in_specs=[
        pl.BlockSpec((1, 1), lambda i: (0, 0)),
        _row_spec, _row_spec, _row_spec, _one_spec, _one_spec,
        pl.BlockSpec((_D, _D), lambda i: (0, 0)),
        pl.BlockSpec((1, _D), lambda i: (0, 0)),
        pl.BlockSpec((_D, 1), lambda i: (0, 0)),
        pl.BlockSpec((_D, 1), lambda i: (0, 0)),
    ],
    out_specs=[_row_spec, _one_spec, _one_spec],
    out_shape=[
        jax.ShapeDtypeStruct((_NP, _D), jnp.float32),
        jax.ShapeDtypeStruct((_NP, 1), jnp.float32),
        jax.ShapeDtypeStruct((_NP, 1), jnp.float32),
    ],
)

_dense3 = pl.pallas_call(
    _tc3_body,
    grid=(_N // _BLK,),
    in_specs=[_row_spec, _row_spec, _one_spec, _one_spec],
    out_specs=_row_spec,
    out_shape=jax.ShapeDtypeStruct((_N, _D), jnp.float32),
)


def _seg_totals(k, v):
    i32 = jnp.int32
    lane = lax.iota(i32, 16)
    prev = k.at[jnp.maximum(lane - 1, 0)].get(mode="promise_in_bounds")
    nxt = k.at[jnp.minimum(lane + 1, 15)].get(mode="promise_in_bounds")
    m_start = (k != prev) | (lane == 0)
    m_last = (k != nxt) | (lane == 15)
    c = plsc.cumsum(v)
    ec = c - v
    ff = plsc.cummax(jnp.where(m_start, ec, 0.0))
    return c - ff, m_last


def _edge_body(xp_hbm, ei_hbm, als_hbm, ald_hbm,
               num_hbm, den_hbm,
               src_v, dst_v, als_v, ald_v, rows2_v, srows_v, exb_v,
               srcb_v, dstl_v, den_v, idx_v, bkt_v, acc_sh, den_sh,
               sem0, sem1):
    cc_ = lax.axis_index("c")
    s = lax.axis_index("s")
    w = s * _NC + cc_

    pltpu.sync_copy(ei_hbm.at[pl.ds(w * _EPW, _EPW)], src_v)
    pltpu.sync_copy(ei_hbm.at[pl.ds(_E + w * _EPW, _EPW)], dst_v)
    pltpu.sync_copy(als_hbm, als_v)
    pltpu.sync_copy(ald_hbm, ald_v)

    zeros16 = jnp.zeros((16,), jnp.float32)
    izeros16 = jnp.zeros((16,), jnp.int32)

    def zrow(r, carry):
        for k in range(_D // 16):
            srows_v[r, pl.ds(k * 16, 16)] = zeros16
        return carry

    lax.fori_loop(0, 64, zrow, 0)

    def zden(r, carry):
        for k in range(_D // 16):
            den_v[r, pl.ds(k * 16, 16)] = zeros16
        return carry

    lax.fori_loop(0, _DR, zden, 0)

    for g in range(_DR // 16):
        idx_v[pl.ds(g * 16, 16)] = lax.iota(jnp.int32, 16) + g * 16

    for g in range(3 * _CHB // 16):
        bkt_v[pl.ds(_EPW + g * 16, 16)] = izeros16

    for t in range(_RPS // 64):
        pltpu.sync_copy(srows_v, acc_sh.at[pl.ds(s * _RPS + t * 64, 64)])

    @pl.when(s < _DR // 8)
    def _zero_den_sh():
        pltpu.sync_copy(srows_v.at[pl.ds(0, 8)], den_sh.at[pl.ds(s * 8, 8)])

    plsc.subcore_barrier()

    def cntf(j, c):
        eb = j * _CH
        c0, c1, c2, c3, c4 = c
        for g in range(_CH // 16):
            di = dst_v[pl.ds(eb + g * 16, 16)]
            b16 = lax.shift_right_logical(di, 11)
            c0 += plsc.all_reduce_population_count(b16 == 0)[0]
            c1 += plsc.all_reduce_population_count(b16 == 1)[0]
            c2 += plsc.all_reduce_population_count(b16 == 2)[0]
            c3 += plsc.all_reduce_population_count(b16 == 3)[0]
            c4 += plsc.all_reduce_population_count(b16 == 4)[0]
        return (c0, c1, c2, c3, c4)

    zi = jnp.int32(0)
    c0, c1, c2, c3, c4 = lax.fori_loop(0, _NCHUNK, cntf,
                                       (zi, zi, zi, zi, zi))
    o1 = c0
    o2 = o1 + c1
    o3 = o2 + c2
    o4 = o3 + c3

    def appf(j, pos):
        eb = j * _CH
        p0, p1, p2, p3, p4 = pos
        for g in range(_CH // 16):
            si = src_v[pl.ds(eb + g * 16, 16)]
            di = dst_v[pl.ds(eb + g * 16, 16)]
            a_s = plsc.load_gather(als_v, [si])
            a_d = plsc.load_gather(ald_v, [di])
            e = a_s + a_d
            e = jnp.where(e >= 0.0, e, 0.2 * e)
            ex = jnp.exp(e)
            dk, dv = plsc.sort_key_val(di, ex)
            tot, m_last = _seg_totals(dk, dv)
            plsc.addupdate_scatter(
                den_v,
                [lax.shift_right_logical(dk, 7),
                 lax.bitwise_and(dk, jnp.full((16,), 127, jnp.int32))],
                tot, mask=m_last)
            b16 = lax.shift_right_logical(di, 11)
            pk = si * 4096 + (di - b16 * _NPH)
            m0 = b16 == 0
            m1 = b16 == 1
            m2 = b16 == 2
            m3 = b16 == 3
            m4 = b16 == 4
            plsc.store_compressed(bkt_v.at[pl.ds(p0, 16)], pk, mask=m0)
            plsc.store_compressed(bkt_v.at[pl.ds(p1, 16)], pk, mask=m1)
            plsc.store_compressed(bkt_v.at[pl.ds(p2, 16)], pk, mask=m2)
            plsc.store_compressed(bkt_v.at[pl.ds(p3, 16)], pk, mask=m3)
            plsc.store_compressed(bkt_v.at[pl.ds(p4, 16)], pk, mask=m4)
            p0 += plsc.all_reduce_population_count(m0)[0]
            p1 += plsc.all_reduce_population_count(m1)[0]
            p2 += plsc.all_reduce_population_count(m2)[0]
            p3 += plsc.all_reduce_population_count(m3)[0]
            p4 += plsc.all_reduce_population_count(m4)[0]
        return (p0, p1, p2, p3, p4)

    lax.fori_loop(0, _NCHUNK, appf, (zi, o1, o2, o3, o4))

    pltpu.sync_copy(den_v, den_sh.at[idx_v], add=True)
    plsc.subcore_barrier()

    @pl.when(s < _DR // 8)
    def _write_den():
        dsl = pl.ds(s * 8, 8)
        pltpu.sync_copy(den_sh.at[dsl], den_hbm.at[cc_, dsl])

    lax.fori_loop(0, _DR, zden, 0)

    starts = (zi, o1, o2, o3, o4)
    counts = (c0, c1, c2, c3, c4)
    mask4095 = jnp.full((16,), 4095, jnp.int32)

    for p in range(_NPASS):
        base = p * _NPH
        off = starts[p]
        cnt = counts[p]
        npairs = (cnt + 2 * _CHB - 1) // (2 * _CHB)

        def stage(jj, b):
            cb = off + jj * _CHB

            def sg(g, carry):
                pk = bkt_v[pl.ds(cb + g * 16, 16)]
                si = lax.shift_right_logical(pk, 12)
                dl = lax.bitwise_and(pk, mask4095)
                a_s = plsc.load_gather(als_v, [si])
                a_d = plsc.load_gather(ald_v, [dl + base])
                e = a_s + a_d
                e = jnp.where(e >= 0.0, e, 0.2 * e)
                ex = jnp.exp(e)
                lane = lax.iota(jnp.int32, 16) + (jj * _CHB + g * 16)
                exb_v[b, pl.ds(g * 16, 16)] = jnp.where(lane < cnt, ex, 0.0)
                srcb_v[b, pl.ds(g * 16, 16)] = si
                dstl_v[b, pl.ds(g * 16, 16)] = dl
                return carry

            lax.fori_loop(0, _CHB // 16, sg, 0)

        def gwait(b, sem_b):
            pltpu.make_async_copy(xp_hbm.at[srcb_v.at[b]],
                                  rows2_v.at[b], sem_b).wait()

        def proc(b):
            def sgroup(g2, icarry):
                exv = exb_v[b, pl.ds(g2 * 16, 16)]
                for l in range(16):
                    sc = exv[l]
                    r = g2 * 16 + l
                    for k in range(_D // 16):
                        rows2_v[b, r, pl.ds(k * 16, 16)] = (
                            rows2_v[b, r, pl.ds(k * 16, 16)] * sc)
                return icarry

            lax.fori_loop(0, _CHB // 16, sgroup, 0)
            pltpu.sync_copy(rows2_v.at[b], acc_sh.at[dstl_v.at[b]], add=True)

        stage(0, 0)
        pltpu.async_copy(xp_hbm.at[srcb_v.at[0]], rows2_v.at[0], sem0)

        def pair(j2, carry):
            je = 2 * j2
            stage(je + 1, 1)
            pltpu.async_copy(xp_hbm.at[srcb_v.at[1]], rows2_v.at[1], sem1)
            gwait(0, sem0)
            proc(0)
            stage(je + 2, 0)
            pltpu.async_copy(xp_hbm.at[srcb_v.at[0]], rows2_v.at[0], sem0)
            gwait(1, sem1)
            proc(1)
            return carry

        lax.fori_loop(0, npairs, pair, 0)
        gwait(0, sem0)
        plsc.subcore_barrier()

        for t in range(_RPS // 64):
            sl = pl.ds(s * _RPS + t * 64, 64)
            osl = pl.ds(base + s * _RPS + t * 64, 64)
            pltpu.sync_copy(acc_sh.at[sl], num_hbm.at[cc_, osl])
        if p < _NPASS - 1:
            for t in range(_RPS // 64):
                sl = pl.ds(s * _RPS + t * 64, 64)
                pltpu.sync_copy(den_v.at[pl.ds(0, 64)], acc_sh.at[sl])
            plsc.subcore_barrier()


_edge = pl.kernel(
    _edge_body,
    out_type=[
        jax.ShapeDtypeStruct((_NC, _NPA, _D), jnp.float32),
        jax.ShapeDtypeStruct((_NC, _DR, _D), jnp.float32),
    ],
    mesh=plsc.VectorSubcoreMesh(core_axis_name="c", subcore_axis_name="s"),
    compiler_params=pltpu.CompilerParams(needs_layout_passes=False),
    scratch_types=[
        pltpu.VMEM((_EPW,), jnp.int32),
        pltpu.VMEM((_EPW,), jnp.int32),
        pltpu.VMEM((_NP,), jnp.float32),
        pltpu.VMEM((_NP,), jnp.float32),
        pltpu.VMEM((2, _CHB, _D), jnp.float32),
        pltpu.VMEM((64, _D), jnp.float32),
        pltpu.VMEM((2, _CHB), jnp.float32),
        pltpu.VMEM((2, _CHB), jnp.int32),
        pltpu.VMEM((2, _CHB), jnp.int32),
        pltpu.VMEM((_DR, _D), jnp.float32),
        pltpu.VMEM((_DR,), jnp.int32),
        pltpu.VMEM((_EPW + 3 * _CHB,), jnp.int32),
        pltpu.VMEM_SHARED((_NPH, _D), jnp.float32),
        pltpu.VMEM_SHARED((_DR, _D), jnp.float32),
        pltpu.SemaphoreType.DMA,
        pltpu.SemaphoreType.DMA,
    ],
)


def kernel(x, edge_index, W1, b1, as1, ad1, Wk1, bk1, q1,
           W2, b2, as2, ad2, Wk2, bk2, q2):
    ei_flat = edge_index.reshape(-1)
    ws = jnp.stack([W1, W2])
    bs = jnp.stack([b1.reshape(1, _D), b2.reshape(1, _D)])
    avs = jnp.stack([as1.reshape(_D, 1), as2.reshape(_D, 1)])
    advs = jnp.stack([ad1.reshape(_D, 1), ad2.reshape(_D, 1)])
    preds = jnp.array([[[1.0]], [[0.0]]], dtype=jnp.float32)

    num0 = jnp.zeros((_NC, _NPA, _D), jnp.float32)
    den0 = jnp.zeros((_NC, _DR, _D), jnp.float32)

    def body(carry, layer):
        num, den = carry
        pred, w, b, av, adv = layer
        xp, als, ald = _tcmain(pred, x, num[0], num[1],
                               den[0].reshape(_NP, 1),
                               den[1].reshape(_NP, 1),
                               w, b, av, adv)
        num2, den2 = _edge(xp, ei_flat, als.reshape(-1), ald.reshape(-1))
        return (num2, den2), 0

    (num_f, den_f), _ = lax.scan(body, (num0, den0),
                                 (preds, ws, bs, avs, advs))
    return _dense3(num_f[0], num_f[1],
                   den_f[0].reshape(_NP, 1), den_f[1].reshape(_NP, 1))


# speedup vs baseline: 1.3365x; 1.0852x over previous
import jax
import jax.numpy as jnp
from jax import lax
from jax.experimental import pallas as pl
from jax.experimental.pallas import tpu as pltpu
from jax.experimental.pallas import tpu_sc as plsc

_N = 10000
_E = 320000
_D = 128
_NC = 2
_NS = 16
_NW = _NC * _NS
_EPW = _E // _NW
_CH = 80
_CHB = 80
_NCHUNK = _EPW // _CH
_NP = 10240
_NPASS = 4
_NPH = 2560
_NPA = _NPASS * _NPH
_RPS = _NPH // _NS
_DR = _NP // _D
_BLK = 2000


def _tcmain_body(p_ref, x_ref, n0_ref, n1_ref, d0_ref, d1_ref,
                 w_ref, b_ref, av_ref, adv_ref,
                 xp_ref, als_ref, ald_ref):
    num = n0_ref[...] + n1_ref[...]
    den = d0_ref[...] + d1_ref[...]
    hprev = jnp.maximum(num / (den + 1e-16), 0.0)
    h = jnp.where(p_ref[0, 0] > 0.0, x_ref[...], hprev)
    xp = jnp.dot(h, w_ref[...],
                 preferred_element_type=jnp.float32) + b_ref[...]
    xp_ref[...] = xp
    als_ref[...] = jnp.dot(xp, av_ref[...], preferred_element_type=jnp.float32)
    ald_ref[...] = jnp.dot(xp, adv_ref[...], preferred_element_type=jnp.float32)


def _tc3_body(n0_ref, n1_ref, d0_ref, d1_ref, o_ref):
    num = n0_ref[...] + n1_ref[...]
    den = d0_ref[...] + d1_ref[...]
    o_ref[...] = jnp.maximum(num / (den + 1e-16), 0.0)


_row_spec = pl.BlockSpec((_BLK, _D), lambda i: (i, 0))
_one_spec = pl.BlockSpec((_BLK, 1), lambda i: (i, 0))

_tcmain = pl.pallas_call(
    _tcmain_body,
    grid=(_N // _BLK,),
    in_specs=[
        pl.BlockSpec((1, 1), lambda i: (0, 0)),
        _row_spec, _row_spec, _row_spec, _one_spec, _one_spec,
        pl.BlockSpec((_D, _D), lambda i: (0, 0)),
        pl.BlockSpec((1, _D), lambda i: (0, 0)),
        pl.BlockSpec((_D, 1), lambda i: (0, 0)),
        pl.BlockSpec((_D, 1), lambda i: (0, 0)),
    ],
    out_specs=[_row_spec, _one_spec, _one_spec],
    out_shape=[
        jax.ShapeDtypeStruct((_NP, _D), jnp.float32),
        jax.ShapeDtypeStruct((_NP, 1), jnp.float32),
        jax.ShapeDtypeStruct((_NP, 1), jnp.float32),
    ],
)

_dense3 = pl.pallas_call(
    _tc3_body,
    grid=(_N // _BLK,),
    in_specs=[_row_spec, _row_spec, _one_spec, _one_spec],
    out_specs=_row_spec,
    out_shape=jax.ShapeDtypeStruct((_N, _D), jnp.float32),
)


def _seg_totals(k, v):
    i32 = jnp.int32
    lane = lax.iota(i32, 16)
    prev = k.at[jnp.maximum(lane - 1, 0)].get(mode="promise_in_bounds")
    nxt = k.at[jnp.minimum(lane + 1, 15)].get(mode="promise_in_bounds")
    m_start = (k != prev) | (lane == 0)
    m_last = (k != nxt) | (lane == 15)
    c = plsc.cumsum(v)
    ec = c - v
    ff = plsc.cummax(jnp.where(m_start, ec, 0.0))
    return c - ff, m_last


def _edge_body(xp_hbm, ei_hbm, als_hbm, ald_hbm,
               num_hbm, den_hbm,
               src_v, dst_v, als_v, ald_v, rows2_v, srows_v, exb_v,
               srcb_v, dstl_v, den_v, idx_v, bkt_v, acc_sh, den_sh,
               sem0, sem1):
    cc_ = lax.axis_index("c")
    s = lax.axis_index("s")
    w = s * _NC + cc_

    pltpu.sync_copy(ei_hbm.at[pl.ds(w * _EPW, _EPW)], src_v)
    pltpu.sync_copy(ei_hbm.at[pl.ds(_E + w * _EPW, _EPW)], dst_v)
    pltpu.sync_copy(als_hbm, als_v)
    pltpu.sync_copy(ald_hbm, ald_v)

    zeros16 = jnp.zeros((16,), jnp.float32)
    izeros16 = jnp.zeros((16,), jnp.int32)

    def zrow(r, carry):
        for k in range(_D // 16):
            srows_v[r, pl.ds(k * 16, 16)] = zeros16
        return carry

    lax.fori_loop(0, 64, zrow, 0)

    def zden(r, carry):
        for k in range(_D // 16):
            den_v[r, pl.ds(k * 16, 16)] = zeros16
        return carry

    lax.fori_loop(0, _DR, zden, 0)

    for g in range(_DR // 16):
        idx_v[pl.ds(g * 16, 16)] = lax.iota(jnp.int32, 16) + g * 16

    for g in range(3 * _CHB // 16):
        bkt_v[pl.ds(_EPW + g * 16, 16)] = izeros16

    for t in range(_RPS // 64):
        pltpu.sync_copy(srows_v, acc_sh.at[pl.ds(s * _RPS + t * 64, 64)])

    @pl.when(s < _DR // 8)
    def _zero_den_sh():
        pltpu.sync_copy(srows_v.at[pl.ds(0, 8)], den_sh.at[pl.ds(s * 8, 8)])

    plsc.subcore_barrier()

    def cntf(j, c):
        eb = j * _CH
        c0, c1, c2, c3 = c
        for g in range(_CH // 16):
            di = dst_v[pl.ds(eb + g * 16, 16)]
            b16 = lax.shift_right_logical(di * 26215, 26)
            c0 += plsc.all_reduce_population_count(b16 == 0)[0]
            c1 += plsc.all_reduce_population_count(b16 == 1)[0]
            c2 += plsc.all_reduce_population_count(b16 == 2)[0]
            c3 += plsc.all_reduce_population_count(b16 == 3)[0]
        return (c0, c1, c2, c3)

    zi = jnp.int32(0)
    c0, c1, c2, c3 = lax.fori_loop(0, _NCHUNK, cntf, (zi, zi, zi, zi))
    o1 = c0
    o2 = o1 + c1
    o3 = o2 + c2

    def appf(j, pos):
        eb = j * _CH
        p0, p1, p2, p3 = pos
        for g in range(_CH // 16):
            si = src_v[pl.ds(eb + g * 16, 16)]
            di = dst_v[pl.ds(eb + g * 16, 16)]
            a_s = plsc.load_gather(als_v, [si])
            a_d = plsc.load_gather(ald_v, [di])
            e = a_s + a_d
            e = jnp.where(e >= 0.0, e, 0.2 * e)
            ex = jnp.exp(e)
            dk, dv = plsc.sort_key_val(di, ex)
            tot, m_last = _seg_totals(dk, dv)
            plsc.addupdate_scatter(
                den_v,
                [lax.shift_right_logical(dk, 7),
                 lax.bitwise_and(dk, jnp.full((16,), 127, jnp.int32))],
                tot, mask=m_last)
            b16 = lax.shift_right_logical(di * 26215, 26)
            pk = si * 4096 + (di - b16 * _NPH)
            m0 = b16 == 0
            m1 = b16 == 1
            m2 = b16 == 2
            m3 = b16 == 3
            plsc.store_compressed(bkt_v.at[pl.ds(p0, 16)], pk, mask=m0)
            plsc.store_compressed(bkt_v.at[pl.ds(p1, 16)], pk, mask=m1)
            plsc.store_compressed(bkt_v.at[pl.ds(p2, 16)], pk, mask=m2)
            plsc.store_compressed(bkt_v.at[pl.ds(p3, 16)], pk, mask=m3)
            p0 += plsc.all_reduce_population_count(m0)[0]
            p1 += plsc.all_reduce_population_count(m1)[0]
            p2 += plsc.all_reduce_population_count(m2)[0]
            p3 += plsc.all_reduce_population_count(m3)[0]
        return (p0, p1, p2, p3)

    lax.fori_loop(0, _NCHUNK, appf, (zi, o1, o2, o3))

    pltpu.sync_copy(den_v, den_sh.at[idx_v], add=True)
    plsc.subcore_barrier()

    @pl.when(s < _DR // 8)
    def _write_den():
        dsl = pl.ds(s * 8, 8)
        pltpu.sync_copy(den_sh.at[dsl], den_hbm.at[cc_, dsl])

    lax.fori_loop(0, _DR, zden, 0)

    starts = (zi, o1, o2, o3)
    counts = (c0, c1, c2, c3)
    mask4095 = jnp.full((16,), 4095, jnp.int32)

    for p in range(_NPASS):
        base = p * _NPH
        off = starts[p]
        cnt = counts[p]
        npairs = (cnt + 2 * _CHB - 1) // (2 * _CHB)

        def stage(jj, b):
            cb = off + jj * _CHB

            def sg(g, carry):
                pk = bkt_v[pl.ds(cb + g * 16, 16)]
                si = lax.shift_right_logical(pk, 12)
                dl = lax.bitwise_and(pk, mask4095)
                a_s = plsc.load_gather(als_v, [si])
                a_d = plsc.load_gather(ald_v, [dl + base])
                e = a_s + a_d
                e = jnp.where(e >= 0.0, e, 0.2 * e)
                ex = jnp.exp(e)
                lane = lax.iota(jnp.int32, 16) + (jj * _CHB + g * 16)
                exb_v[b, pl.ds(g * 16, 16)] = jnp.where(lane < cnt, ex, 0.0)
                srcb_v[b, pl.ds(g * 16, 16)] = si
                dstl_v[b, pl.ds(g * 16, 16)] = dl
                return carry

            lax.fori_loop(0, _CHB // 16, sg, 0)

        def gwait(b, sem_b):
            pltpu.make_async_copy(xp_hbm.at[srcb_v.at[b]],
                                  rows2_v.at[b], sem_b).wait()

        def proc(b):
            def sgroup(g2, icarry):
                exv = exb_v[b, pl.ds(g2 * 16, 16)]
                for l in range(16):
                    sc = exv[l]
                    r = g2 * 16 + l
                    for k in range(_D // 16):
                        rows2_v[b, r, pl.ds(k * 16, 16)] = (
                            rows2_v[b, r, pl.ds(k * 16, 16)] * sc)
                return icarry

            lax.fori_loop(0, _CHB // 16, sgroup, 0)
            pltpu.sync_copy(rows2_v.at[b], acc_sh.at[dstl_v.at[b]], add=True)

        stage(0, 0)
        pltpu.async_copy(xp_hbm.at[srcb_v.at[0]], rows2_v.at[0], sem0)

        def pair(j2, carry):
            je = 2 * j2
            stage(je + 1, 1)
            pltpu.async_copy(xp_hbm.at[srcb_v.at[1]], rows2_v.at[1], sem1)
            gwait(0, sem0)
            proc(0)
            stage(je + 2, 0)
            pltpu.async_copy(xp_hbm.at[srcb_v.at[0]], rows2_v.at[0], sem0)
            gwait(1, sem1)
            proc(1)
            return carry

        lax.fori_loop(0, npairs, pair, 0)
        gwait(0, sem0)
        plsc.subcore_barrier()

        for t in range(_RPS // 64):
            sl = pl.ds(s * _RPS + t * 64, 64)
            osl = pl.ds(base + s * _RPS + t * 64, 64)
            pltpu.sync_copy(acc_sh.at[sl], num_hbm.at[cc_, osl])
        if p < _NPASS - 1:
            for t in range(_RPS // 64):
                sl = pl.ds(s * _RPS + t * 64, 64)
                pltpu.sync_copy(den_v.at[pl.ds(0, 64)], acc_sh.at[sl])
            plsc.subcore_barrier()


_edge = pl.kernel(
    _edge_body,
    out_type=[
        jax.ShapeDtypeStruct((_NC, _NPA, _D), jnp.float32),
        jax.ShapeDtypeStruct((_NC, _DR, _D), jnp.float32),
    ],
    mesh=plsc.VectorSubcoreMesh(core_axis_name="c", subcore_axis_name="s"),
    compiler_params=pltpu.CompilerParams(needs_layout_passes=False),
    scratch_types=[
        pltpu.VMEM((_EPW,), jnp.int32),
        pltpu.VMEM((_EPW,), jnp.int32),
        pltpu.VMEM((_NP,), jnp.float32),
        pltpu.VMEM((_NP,), jnp.float32),
        pltpu.VMEM((2, _CHB, _D), jnp.float32),
        pltpu.VMEM((64, _D), jnp.float32),
        pltpu.VMEM((2, _CHB), jnp.float32),
        pltpu.VMEM((2, _CHB), jnp.int32),
        pltpu.VMEM((2, _CHB), jnp.int32),
        pltpu.VMEM((_DR, _D), jnp.float32),
        pltpu.VMEM((_DR,), jnp.int32),
        pltpu.VMEM((_EPW + 3 * _CHB,), jnp.int32),
        pltpu.VMEM_SHARED((_NPH, _D), jnp.float32),
        pltpu.VMEM_SHARED((_DR, _D), jnp.float32),
        pltpu.SemaphoreType.DMA,
        pltpu.SemaphoreType.DMA,
    ],
)


def kernel(x, edge_index, W1, b1, as1, ad1, Wk1, bk1, q1,
           W2, b2, as2, ad2, Wk2, bk2, q2):
    ei_flat = edge_index.reshape(-1)
    ws = jnp.stack([W1, W2])
    bs = jnp.stack([b1.reshape(1, _D), b2.reshape(1, _D)])
    avs = jnp.stack([as1.reshape(_D, 1), as2.reshape(_D, 1)])
    advs = jnp.stack([ad1.reshape(_D, 1), ad2.reshape(_D, 1)])
    preds = jnp.array([[[1.0]], [[0.0]]], dtype=jnp.float32)

    num0 = jnp.zeros((_NC, _NPA, _D), jnp.float32)
    den0 = jnp.zeros((_NC, _DR, _D), jnp.float32)

    def body(carry, layer):
        num, den = carry
        pred, w, b, av, adv = layer
        xp, als, ald = _tcmain(pred, x, num[0], num[1],
                               den[0].reshape(_NP, 1),
                               den[1].reshape(_NP, 1),
                               w, b, av, adv)
        num2, den2 = _edge(xp, ei_flat, als.reshape(-1), ald.reshape(-1))
        return (num2, den2), 0

    (num_f, den_f), _ = lax.scan(body, (num0, den0),
                                 (preds, ws, bs, avs, advs))
    return _dense3(num_f[0], num_f[1],
                   den_f[0].reshape(_NP, 1), den_f[1].reshape(_NP, 1))
